# Initial kernel scaffold; baseline (speedup 1.0000x reference)
import jax
import jax.numpy as jnp
from jax.experimental import pallas as pl


def kernel(root_indices, prefix_indices, suffix_indices, ending_indices, root_table, prefix_table, suffix_table, ending_table, proj_w, proj_b, ln_gamma, ln_beta):
    raise NotImplementedError("write your pallas kernel here")


# speedup vs baseline: 8.5294x; 8.5294x over previous
import functools

import jax
import jax.numpy as jnp
from jax.experimental import pallas as pl

_R, _P, _S, _E = 64, 16, 32, 17
_D = 128
_KP = 144
_OFF_P, _OFF_S, _OFF_E = _R, _R + _P, _R + _P + _S


def _premul_body(tabs_ref, w_ref, out_ref):
    out_ref[...] = jnp.dot(
        tabs_ref[...], w_ref[...], preferred_element_type=jnp.float32
    ).astype(jnp.bfloat16)


def _onehot(idx, k):
    cols = jax.lax.broadcasted_iota(jnp.int32, (idx.shape[0], k), 1)
    oh32 = jnp.where(cols == idx[:, None], jnp.float32(1), jnp.float32(0))
    return oh32.astype(jnp.bfloat16)


def _main_body(ri_ref, pi_ref, si_ref, ei_ref, wc_ref, b_ref, g_ref, bb_ref,
               out_ref, *, t):
    ri = ri_ref[0, 0, :]
    pi = pi_ref[0, 0, :]
    si = si_ref[0, 0, :]
    ei = ei_ref[0, 0, :]
    acc = jnp.dot(_onehot(ri, _R), wc_ref[0:_OFF_P],
                  preferred_element_type=jnp.float32)
    acc += jnp.dot(_onehot(pi, _P), wc_ref[_OFF_P:_OFF_S],
                   preferred_element_type=jnp.float32)
    acc += jnp.dot(_onehot(si, _S), wc_ref[_OFF_S:_OFF_E],
                   preferred_element_type=jnp.float32)
    acc += jnp.dot(_onehot(ei, 32), wc_ref[_OFF_E:_OFF_E + 32],
                   preferred_element_type=jnp.float32)
    x = acc + b_ref[0, :][None, :]
    mu = jnp.mean(x, axis=1, keepdims=True)
    xc = x - mu
    var = jnp.mean(xc * xc, axis=1, keepdims=True)
    out_ref[...] = xc * jax.lax.rsqrt(var + 1e-5) * g_ref[0, :][None, :] \
        + bb_ref[0, :][None, :]


def kernel(root_indices, prefix_indices, suffix_indices, ending_indices,
           root_table, prefix_table, suffix_table, ending_table,
           proj_w, proj_b, ln_gamma, ln_beta):
    b, l = root_indices.shape
    n = b * l
    t = 2048
    g = n // t
    assert g * t == n

    tabs = jnp.concatenate([
        root_table, prefix_table, suffix_table, ending_table,
        jnp.zeros((_KP - _OFF_E - _E, _D), jnp.float32),
    ], axis=0)

    wc = pl.pallas_call(
        _premul_body,
        out_shape=jax.ShapeDtypeStruct((_KP, _D), jnp.bfloat16),
    )(tabs, proj_w)

    idx_spec = pl.BlockSpec((1, 1, t), lambda i: (i, 0, 0))
    vec_spec = pl.BlockSpec((1, _D), lambda i: (0, 0))
    out = pl.pallas_call(
        functools.partial(_main_body, t=t),
        grid=(g,),
        in_specs=[
            idx_spec, idx_spec, idx_spec, idx_spec,
            pl.BlockSpec((_KP, _D), lambda i: (0, 0)),
            vec_spec, vec_spec, vec_spec,
        ],
        out_specs=pl.BlockSpec((t, _D), lambda i: (i, 0)),
        out_shape=jax.ShapeDtypeStruct((n, _D), jnp.float32),
    )(
        root_indices.reshape(g, 1, t),
        prefix_indices.reshape(g, 1, t),
        suffix_indices.reshape(g, 1, t),
        ending_indices.reshape(g, 1, t),
        wc,
        proj_b.reshape(1, _D),
        ln_gamma.reshape(1, _D),
        ln_beta.reshape(1, _D),
    )
    return out.reshape(b, l, _D)


# speedup vs baseline: 10.4443x; 1.2245x over previous
import functools

import jax
import jax.numpy as jnp
from jax.experimental import pallas as pl

_R, _P, _S, _E = 64, 16, 32, 17
_D = 128
_KP = 144
_OFF_P, _OFF_S, _OFF_E = _R, _R + _P, _R + _P + _S


def _premul_body(tabs_ref, w_ref, out_ref):
    out_ref[...] = jnp.dot(
        tabs_ref[...], w_ref[...], preferred_element_type=jnp.float32
    ).astype(jnp.bfloat16)


def _main_body(ri_ref, pi_ref, si_ref, ei_ref, wc_ref, b_ref, g_ref, bb_ref,
               out_ref, *, tb, l):
    wc = wc_ref[...]
    bias = b_ref[0, :][None, :]
    gam = g_ref[0, :][None, :]
    bet = bb_ref[0, :][None, :]
    rb = ri_ref[...].astype(jnp.bfloat16)
    pb = (pi_ref[...] + _OFF_P).astype(jnp.bfloat16)
    sb = (si_ref[...] + _OFF_S).astype(jnp.bfloat16)
    eb = (ei_ref[...] + _OFF_E).astype(jnp.bfloat16)
    cols = jax.lax.broadcasted_iota(jnp.int32, (tb, _KP), 1).astype(jnp.bfloat16)
    one = jnp.bfloat16(1)
    zero = jnp.bfloat16(0)
    for j in range(l):
        oh = (jnp.where(cols == rb[:, j:j + 1], one, zero)
              + jnp.where(cols == pb[:, j:j + 1], one, zero)
              + jnp.where(cols == sb[:, j:j + 1], one, zero)
              + jnp.where(cols == eb[:, j:j + 1], one, zero))
        x = jnp.dot(oh, wc, preferred_element_type=jnp.float32) + bias
        mu = jnp.mean(x, axis=1, keepdims=True)
        xc = x - mu
        var = jnp.mean(xc * xc, axis=1, keepdims=True)
        y = xc * jax.lax.rsqrt(var + 1e-5) * gam + bet
        out_ref[:, j, :] = y


def kernel(root_indices, prefix_indices, suffix_indices, ending_indices,
           root_table, prefix_table, suffix_table, ending_table,
           proj_w, proj_b, ln_gamma, ln_beta):
    b, l = root_indices.shape
    tb = 512
    g = b // tb
    assert g * tb == b

    tabs = jnp.concatenate([
        root_table, prefix_table, suffix_table, ending_table,
        jnp.zeros((_KP - _OFF_E - _E, _D), jnp.float32),
    ], axis=0)

    wc = pl.pallas_call(
        _premul_body,
        out_shape=jax.ShapeDtypeStruct((_KP, _D), jnp.bfloat16),
    )(tabs, proj_w)

    idx_spec = pl.BlockSpec((tb, l), lambda i: (i, 0))
    vec_spec = pl.BlockSpec((1, _D), lambda i: (0, 0))
    out = pl.pallas_call(
        functools.partial(_main_body, tb=tb, l=l),
        grid=(g,),
        in_specs=[
            idx_spec, idx_spec, idx_spec, idx_spec,
            pl.BlockSpec((_KP, _D), lambda i: (0, 0)),
            vec_spec, vec_spec, vec_spec,
        ],
        out_specs=pl.BlockSpec((tb, l, _D), lambda i: (i, 0, 0)),
        out_shape=jax.ShapeDtypeStruct((b, l, _D), jnp.float32),
    )(
        root_indices, prefix_indices, suffix_indices, ending_indices,
        wc,
        proj_b.reshape(1, _D),
        ln_gamma.reshape(1, _D),
        ln_beta.reshape(1, _D),
    )
    return out
